# baseline (speedup 1.0000x reference)
import functools

import jax
import jax.numpy as jnp
from jax import lax
from jax.experimental import pallas as pl
from jax.experimental.pallas import tpu as pltpu
from jax.experimental.pallas import tpu_sc as plsc

EMB_DIM = 64
NUM_CORES = 2
NUM_SUBCORES = 16
NUM_WORKERS = NUM_CORES * NUM_SUBCORES
NBUF = 2


@functools.partial(jax.jit, static_argnums=(2, 3))
def _sc_gather(weight, flat_idx, total, chunk):
    rows_per_worker = total // NUM_WORKERS
    nchunks = rows_per_worker // chunk
    mesh = plsc.VectorSubcoreMesh(core_axis_name="c", subcore_axis_name="s")

    @functools.partial(
        pl.kernel,
        out_type=jax.ShapeDtypeStruct((total, EMB_DIM), jnp.float32),
        mesh=mesh,
        scratch_types=(
            [pltpu.VMEM((chunk,), jnp.int32) for _ in range(NBUF)]
            + [pltpu.VMEM((chunk, EMB_DIM), jnp.float32) for _ in range(NBUF)]
            + [pltpu.SemaphoreType.DMA for _ in range(NBUF)]
        ),
        compiler_params=pltpu.CompilerParams(use_tc_tiling_on_sc=False),
    )
    def k(table_hbm, idx_hbm, out_hbm, *scratch):
        idx_bufs = scratch[:NBUF]
        row_bufs = scratch[NBUF:2 * NBUF]
        sems = scratch[2 * NBUF:]
        wid = lax.axis_index("s") * NUM_CORES + lax.axis_index("c")
        base = wid * rows_per_worker

        descs = [None] * NBUF
        for c in range(min(NBUF, nchunks)):
            start = base + c * chunk
            pltpu.sync_copy(idx_hbm.at[pl.ds(start, chunk)], idx_bufs[c])
            descs[c] = pltpu.async_copy(
                table_hbm.at[idx_bufs[c]], row_bufs[c], sems[c])

        for c in range(nchunks):
            b = c % NBUF
            descs[b].wait()
            pltpu.sync_copy(row_bufs[b], out_hbm.at[pl.ds(base + c * chunk, chunk)])
            nxt = c + NBUF
            if nxt < nchunks:
                start = base + nxt * chunk
                pltpu.sync_copy(idx_hbm.at[pl.ds(start, chunk)], idx_bufs[b])
                descs[b] = pltpu.async_copy(
                    table_hbm.at[idx_bufs[b]], row_bufs[b], sems[b])

    return k(weight, flat_idx)


def kernel(indices, weight):
    batch, fields = indices.shape
    flat = indices.reshape(-1).astype(jnp.int32)
    out = _sc_gather(weight, flat, flat.shape[0], 512)
    return out.reshape(batch, fields, EMB_DIM)
